# Initial kernel scaffold; baseline (speedup 1.0000x reference)
import functools

import jax
import jax.numpy as jnp
from jax import lax
from jax.experimental import pallas as pl
from jax.experimental.pallas import tpu as pltpu
from jax.experimental.pallas import tpu_sc as plsc

B = 16
N = 65536
FEAT = 64
INV_T = 1.0 / 0.07

NUM_WORKERS = 32
ROWS = 6 * B
CHUNK = 8192
N_CHUNKS = N // CHUNK


def _qt_kernel(fz_ref, fm_ref, fr_ref, bz_ref, bm_ref, br_ref, out_ref):
    fz = fz_ref[...] * INV_T
    fm = fm_ref[...] * INV_T
    fr = fr_ref[...] * INV_T
    dims = (((1,), (1,)), ((), ()))
    out_ref[0:32, :] = lax.dot_general(
        fz, bz_ref[...], dims, preferred_element_type=jnp.float32)
    out_ref[32:64, :] = lax.dot_general(
        fm, bm_ref[...], dims, preferred_element_type=jnp.float32)
    out_ref[64:96, :] = lax.dot_general(
        fr, br_ref[...], dims, preferred_element_type=jnp.float32)


def _compute_qt(f_z, f_m, f_r, bank_z, bank_m, bank_r):
    blk = 4096
    grid = (N // blk,)
    return pl.pallas_call(
        _qt_kernel,
        grid=grid,
        in_specs=[
            pl.BlockSpec((32, FEAT), lambda i: (0, 0)),
            pl.BlockSpec((32, FEAT), lambda i: (0, 0)),
            pl.BlockSpec((32, FEAT), lambda i: (0, 0)),
            pl.BlockSpec((blk, FEAT), lambda i: (i, 0)),
            pl.BlockSpec((blk, FEAT), lambda i: (i, 0)),
            pl.BlockSpec((blk, FEAT), lambda i: (i, 0)),
        ],
        out_specs=pl.BlockSpec((ROWS, blk), lambda i: (0, i)),
        out_shape=jax.ShapeDtypeStruct((ROWS, N), jnp.float32),
    )(f_z, f_m, f_r, bank_z, bank_m, bank_r)


def _sc_body(qt_hbm, idx_hbm, out_hbm, q_v, idx_v, o_v):
    c = lax.axis_index("c")
    s = lax.axis_index("s")
    wid = s * 2 + c
    b = lax.rem(wid, B)

    for t in range(ROWS // NUM_WORKERS):
        r = t * NUM_WORKERS + wid
        pltpu.sync_copy(qt_hbm.at[r], q_v)
        for ch in range(N_CHUNKS):
            pltpu.sync_copy(idx_hbm.at[b, pl.ds(ch * CHUNK, CHUNK)], idx_v)

            def gather_16(i, _):
                iv = idx_v[pl.ds(i * 16, 16)]
                o_v[pl.ds(i * 16, 16)] = plsc.load_gather(q_v, [iv])
                return 0

            lax.fori_loop(0, CHUNK // 16, gather_16, 0)
            pltpu.sync_copy(o_v, out_hbm.at[r, pl.ds(ch * CHUNK, CHUNK)])


def _sc_gather(qt, idx):
    mesh = plsc.VectorSubcoreMesh(core_axis_name="c", subcore_axis_name="s")
    fn = functools.partial(
        pl.kernel,
        mesh=mesh,
        out_type=jax.ShapeDtypeStruct((ROWS, N), jnp.float32),
        scratch_types=[
            pltpu.VMEM((N,), jnp.float32),
            pltpu.VMEM((CHUNK,), jnp.int32),
            pltpu.VMEM((CHUNK,), jnp.float32),
        ],
    )(_sc_body)
    return fn(qt, idx)


def kernel(fea_f, fea_fenzi, fea_fenmu, y, idx, memory_fringe, memory_fenzi,
           memory_fenmu):
    del y
    idx = idx.astype(jnp.int32)
    f_z = jnp.concatenate([fea_f, fea_fenmu], axis=0)
    f_m = jnp.concatenate([fea_f, fea_fenzi], axis=0)
    f_r = jnp.concatenate([fea_fenzi, fea_fenmu], axis=0)

    qt = _compute_qt(f_z, f_m, f_r, memory_fenzi, memory_fenmu, memory_fringe)
    out = _sc_gather(qt, idx)

    f_fenzi = out[0:16, :, None]
    fenmu_fenzi = out[16:32, :, None]
    f_fenmu = out[32:48, :, None]
    fenzi_fenmu = out[48:64, :, None]
    fenzi_f = out[64:80, :, None]
    fenmu_f = out[80:96, :, None]
    return (f_fenzi, f_fenmu, fenzi_f, fenzi_fenmu, fenmu_f, fenmu_fenzi)


# speedup vs baseline: 36.0270x; 36.0270x over previous
import functools

import jax
import jax.numpy as jnp
from jax import lax
from jax.experimental import pallas as pl
from jax.experimental.pallas import tpu as pltpu
from jax.experimental.pallas import tpu_sc as plsc

B = 16
N = 65536
FEAT = 64
INV_T = 1.0 / 0.07

NUM_WORKERS = 32
ROWS = 6 * B
CHUNK = 8192
N_CHUNKS = N // CHUNK


def _qt_kernel(fz_ref, fm_ref, fr_ref, bz_ref, bm_ref, br_ref, out_ref):
    fz = fz_ref[...] * INV_T
    fm = fm_ref[...] * INV_T
    fr = fr_ref[...] * INV_T
    dims = (((1,), (1,)), ((), ()))
    out_ref[0:32, :] = lax.dot_general(
        fz, bz_ref[...], dims, preferred_element_type=jnp.float32)
    out_ref[32:64, :] = lax.dot_general(
        fm, bm_ref[...], dims, preferred_element_type=jnp.float32)
    out_ref[64:96, :] = lax.dot_general(
        fr, br_ref[...], dims, preferred_element_type=jnp.float32)


def _compute_qt(f_z, f_m, f_r, bank_z, bank_m, bank_r):
    blk = 4096
    grid = (N // blk,)
    return pl.pallas_call(
        _qt_kernel,
        grid=grid,
        in_specs=[
            pl.BlockSpec((32, FEAT), lambda i: (0, 0)),
            pl.BlockSpec((32, FEAT), lambda i: (0, 0)),
            pl.BlockSpec((32, FEAT), lambda i: (0, 0)),
            pl.BlockSpec((blk, FEAT), lambda i: (i, 0)),
            pl.BlockSpec((blk, FEAT), lambda i: (i, 0)),
            pl.BlockSpec((blk, FEAT), lambda i: (i, 0)),
        ],
        out_specs=pl.BlockSpec((ROWS, blk), lambda i: (0, i)),
        out_shape=jax.ShapeDtypeStruct((ROWS, N), jnp.float32),
    )(f_z, f_m, f_r, bank_z, bank_m, bank_r)


def _sc_body(qt_hbm, idx_hbm, out_hbm, q_v, idx_v, o_v):
    c = lax.axis_index("c")
    s = lax.axis_index("s")
    wid = s * 2 + c
    b = lax.rem(wid, B)

    for t in range(ROWS // NUM_WORKERS):
        r = t * NUM_WORKERS + wid
        pltpu.sync_copy(qt_hbm.at[r], q_v)
        for ch in range(N_CHUNKS):
            pltpu.sync_copy(idx_hbm.at[b, pl.ds(ch * CHUNK, CHUNK)], idx_v)

            def gather_16(i, _):
                iv = idx_v[pl.ds(i * 16, 16)]
                o_v[pl.ds(i * 16, 16)] = plsc.load_gather(q_v, [iv])
                return 0

            lax.fori_loop(0, CHUNK // 16, gather_16, 0)
            pltpu.sync_copy(o_v, out_hbm.at[r, pl.ds(ch * CHUNK, CHUNK)])


def _sc_gather(qt, idx):
    mesh = plsc.VectorSubcoreMesh(core_axis_name="c", subcore_axis_name="s")
    fn = functools.partial(
        pl.kernel,
        mesh=mesh,
        out_type=jax.ShapeDtypeStruct((ROWS, N), jnp.float32),
        scratch_types=[
            pltpu.VMEM((N,), jnp.float32),
            pltpu.VMEM((CHUNK,), jnp.int32),
            pltpu.VMEM((CHUNK,), jnp.float32),
        ],
        compiler_params=pltpu.CompilerParams(needs_layout_passes=False),
    )(_sc_body)
    return fn(qt, idx)


def kernel(fea_f, fea_fenzi, fea_fenmu, y, idx, memory_fringe, memory_fenzi,
           memory_fenmu):
    del y
    idx = idx.astype(jnp.int32)
    f_z = jnp.concatenate([fea_f, fea_fenmu], axis=0)
    f_m = jnp.concatenate([fea_f, fea_fenzi], axis=0)
    f_r = jnp.concatenate([fea_fenzi, fea_fenmu], axis=0)

    qt = _compute_qt(f_z, f_m, f_r, memory_fenzi, memory_fenmu, memory_fringe)
    out = _sc_gather(qt, idx)

    f_fenzi = out[0:16, :, None]
    fenmu_fenzi = out[16:32, :, None]
    f_fenmu = out[32:48, :, None]
    fenzi_fenmu = out[48:64, :, None]
    fenzi_f = out[64:80, :, None]
    fenmu_f = out[80:96, :, None]
    return (f_fenzi, f_fenmu, fenzi_f, fenzi_fenmu, fenmu_f, fenmu_fenzi)


# speedup vs baseline: 51.0496x; 1.4170x over previous
import functools

import jax
import jax.numpy as jnp
from jax import lax
from jax.experimental import pallas as pl
from jax.experimental.pallas import tpu as pltpu
from jax.experimental.pallas import tpu_sc as plsc

B = 16
N = 65536
FEAT = 64
INV_T = 1.0 / 0.07

NUM_WORKERS = 32
ROWS = 6 * B
ROUNDS = ROWS // NUM_WORKERS
CHUNK = 8192
N_CHUNKS = N // CHUNK


def _qt_kernel(fz_ref, fm_ref, fr_ref, bz_ref, bm_ref, br_ref, out_ref):
    fz = fz_ref[...] * INV_T
    fm = fm_ref[...] * INV_T
    fr = fr_ref[...] * INV_T
    dims = (((1,), (1,)), ((), ()))
    out_ref[0:32, :] = lax.dot_general(
        fz, bz_ref[...], dims, preferred_element_type=jnp.float32)
    out_ref[32:64, :] = lax.dot_general(
        fm, bm_ref[...], dims, preferred_element_type=jnp.float32)
    out_ref[64:96, :] = lax.dot_general(
        fr, br_ref[...], dims, preferred_element_type=jnp.float32)


def _compute_qt(f_z, f_m, f_r, bank_z, bank_m, bank_r):
    blk = 4096
    grid = (N // blk,)
    return pl.pallas_call(
        _qt_kernel,
        grid=grid,
        in_specs=[
            pl.BlockSpec((32, FEAT), lambda i: (0, 0)),
            pl.BlockSpec((32, FEAT), lambda i: (0, 0)),
            pl.BlockSpec((32, FEAT), lambda i: (0, 0)),
            pl.BlockSpec((blk, FEAT), lambda i: (i, 0)),
            pl.BlockSpec((blk, FEAT), lambda i: (i, 0)),
            pl.BlockSpec((blk, FEAT), lambda i: (i, 0)),
        ],
        out_specs=pl.BlockSpec((ROWS, blk), lambda i: (0, i)),
        out_shape=jax.ShapeDtypeStruct((ROWS, N), jnp.float32),
    )(f_z, f_m, f_r, bank_z, bank_m, bank_r)


def _sc_body(qt_hbm, idx_hbm, o0, o1, o2, o3, o4, o5,
             q_v, idx_v, o_v, sem_q, sem_i0, sem_i1, sem_o0, sem_o1):
    c = lax.axis_index("c")
    s = lax.axis_index("s")
    wid = s * 2 + c
    b = lax.rem(wid, B)
    hi = wid >= B
    outs_lo = (o0, o1, o2)
    outs_hi = (o5, o3, o4)
    idx_sems = (sem_i0, sem_i1)
    out_sems = (sem_o0, sem_o1)

    def idx_copy(ch, bf):
        return pltpu.make_async_copy(
            idx_hbm.at[b, pl.ds(ch * CHUNK, CHUNK)], idx_v.at[bf],
            idx_sems[bf])

    out_pending = [False, False]
    for t in range(ROUNDS):
        r = t * NUM_WORKERS + wid
        pltpu.sync_copy(qt_hbm.at[r], q_v)
        for ch in range(N_CHUNKS):
            bf = ch % 2
            if ch == 0:
                idx_copy(0, 0).start()
            cur = idx_copy(ch, bf)
            cur.wait()
            if ch + 1 < N_CHUNKS:
                idx_copy(ch + 1, (ch + 1) % 2).start()
            if out_pending[bf]:
                pltpu.make_async_copy(
                    qt_hbm.at[0, pl.ds(0, CHUNK)], o_v.at[bf],
                    out_sems[bf]).wait()

            @plsc.parallel_loop(0, CHUNK, 16, unroll=8)
            def _gather(i):
                iv = idx_v[bf, pl.ds(i, 16)]
                o_v[bf, pl.ds(i, 16)] = plsc.load_gather(q_v, [iv])

            col = pl.ds(ch * CHUNK, CHUNK)

            @pl.when(jnp.logical_not(hi))
            def _():
                pltpu.async_copy(o_v.at[bf], outs_lo[t].at[b, col],
                                 out_sems[bf])

            @pl.when(hi)
            def _():
                pltpu.async_copy(o_v.at[bf], outs_hi[t].at[b, col],
                                 out_sems[bf])

            out_pending[bf] = True
    for bf in range(2):
        if out_pending[bf]:
            pltpu.make_async_copy(
                qt_hbm.at[0, pl.ds(0, CHUNK)], o_v.at[bf],
                out_sems[bf]).wait()


def _sc_gather(qt, idx):
    mesh = plsc.VectorSubcoreMesh(core_axis_name="c", subcore_axis_name="s")
    out_t = jax.ShapeDtypeStruct((B, N), jnp.float32)
    fn = functools.partial(
        pl.kernel,
        mesh=mesh,
        out_type=(out_t,) * 6,
        scratch_types=[
            pltpu.VMEM((N,), jnp.float32),
            pltpu.VMEM((2, CHUNK), jnp.int32),
            pltpu.VMEM((2, CHUNK), jnp.float32),
            pltpu.SemaphoreType.DMA,
            pltpu.SemaphoreType.DMA,
            pltpu.SemaphoreType.DMA,
            pltpu.SemaphoreType.DMA,
            pltpu.SemaphoreType.DMA,
        ],
        compiler_params=pltpu.CompilerParams(needs_layout_passes=False),
    )(_sc_body)
    return fn(qt, idx)


def kernel(fea_f, fea_fenzi, fea_fenmu, y, idx, memory_fringe, memory_fenzi,
           memory_fenmu):
    del y
    idx = idx.astype(jnp.int32)
    f_z = jnp.concatenate([fea_f, fea_fenmu], axis=0)
    f_m = jnp.concatenate([fea_f, fea_fenzi], axis=0)
    f_r = jnp.concatenate([fea_fenzi, fea_fenmu], axis=0)

    qt = _compute_qt(f_z, f_m, f_r, memory_fenzi, memory_fenmu, memory_fringe)
    f_fenzi, f_fenmu, fenzi_f, fenzi_fenmu, fenmu_f, fenmu_fenzi = (
        _sc_gather(qt, idx))

    return (f_fenzi[..., None], f_fenmu[..., None], fenzi_f[..., None],
            fenzi_fenmu[..., None], fenmu_f[..., None],
            fenmu_fenzi[..., None])


# speedup vs baseline: 83.4970x; 1.6356x over previous
import functools

import jax
import jax.numpy as jnp
from jax import lax
from jax.experimental import pallas as pl
from jax.experimental.pallas import tpu as pltpu
from jax.experimental.pallas import tpu_sc as plsc

B = 16
N = 65536
FEAT = 64
INV_T = 1.0 / 0.07

NUM_WORKERS = 32
ROWS = 6 * B
ROUNDS = ROWS // NUM_WORKERS
CHUNK = 8192
N_CHUNKS = N // CHUNK


def _qt_kernel(fz_ref, fm_ref, fr_ref, bz_ref, bm_ref, br_ref, out_ref):
    fz = fz_ref[...] * INV_T
    fm = fm_ref[...] * INV_T
    fr = fr_ref[...] * INV_T
    dims = (((1,), (0,)), ((), ()))
    out_ref[0:32, :] = lax.dot_general(
        fz, bz_ref[...], dims, preferred_element_type=jnp.float32)
    out_ref[32:64, :] = lax.dot_general(
        fm, bm_ref[...], dims, preferred_element_type=jnp.float32)
    out_ref[64:96, :] = lax.dot_general(
        fr, br_ref[...], dims, preferred_element_type=jnp.float32)


def _compute_qt(f_z, f_m, f_r, bank_zt, bank_mt, bank_rt):
    blk = 4096
    grid = (N // blk,)
    return pl.pallas_call(
        _qt_kernel,
        grid=grid,
        in_specs=[
            pl.BlockSpec((32, FEAT), lambda i: (0, 0)),
            pl.BlockSpec((32, FEAT), lambda i: (0, 0)),
            pl.BlockSpec((32, FEAT), lambda i: (0, 0)),
            pl.BlockSpec((FEAT, blk), lambda i: (0, i)),
            pl.BlockSpec((FEAT, blk), lambda i: (0, i)),
            pl.BlockSpec((FEAT, blk), lambda i: (0, i)),
        ],
        out_specs=pl.BlockSpec((ROWS, blk), lambda i: (0, i)),
        out_shape=jax.ShapeDtypeStruct((ROWS, N), jnp.float32),
    )(f_z, f_m, f_r, bank_zt, bank_mt, bank_rt)


def _sc_body(qt_hbm, idx_hbm, o0, o1, o2, o3, o4, o5,
             q_v, idx_v, o_v, sem_q, sem_i0, sem_i1, sem_o0, sem_o1):
    c = lax.axis_index("c")
    s = lax.axis_index("s")
    wid = s * 2 + c
    b = lax.rem(wid, B)
    hi = wid >= B
    outs_lo = (o0, o1, o2)
    outs_hi = (o5, o3, o4)
    idx_sems = (sem_i0, sem_i1)
    out_sems = (sem_o0, sem_o1)

    def idx_copy(ch, bf):
        return pltpu.make_async_copy(
            idx_hbm.at[b, pl.ds(ch * CHUNK, CHUNK)], idx_v.at[bf],
            idx_sems[bf])

    out_pending = [False, False]
    for t in range(ROUNDS):
        r = t * NUM_WORKERS + wid
        pltpu.sync_copy(qt_hbm.at[r], q_v)
        for ch in range(N_CHUNKS):
            bf = ch % 2
            if ch == 0:
                idx_copy(0, 0).start()
            cur = idx_copy(ch, bf)
            cur.wait()
            if ch + 1 < N_CHUNKS:
                idx_copy(ch + 1, (ch + 1) % 2).start()
            if out_pending[bf]:
                pltpu.make_async_copy(
                    qt_hbm.at[0, pl.ds(0, CHUNK)], o_v.at[bf],
                    out_sems[bf]).wait()

            @plsc.parallel_loop(0, CHUNK, 16, unroll=8)
            def _gather(i):
                iv = idx_v[bf, pl.ds(i, 16)]
                o_v[bf, pl.ds(i, 16)] = plsc.load_gather(q_v, [iv])

            col = pl.ds(ch * CHUNK, CHUNK)

            @pl.when(jnp.logical_not(hi))
            def _():
                pltpu.async_copy(o_v.at[bf], outs_lo[t].at[b, col],
                                 out_sems[bf])

            @pl.when(hi)
            def _():
                pltpu.async_copy(o_v.at[bf], outs_hi[t].at[b, col],
                                 out_sems[bf])

            out_pending[bf] = True
    for bf in range(2):
        if out_pending[bf]:
            pltpu.make_async_copy(
                qt_hbm.at[0, pl.ds(0, CHUNK)], o_v.at[bf],
                out_sems[bf]).wait()


def _sc_gather(qt, idx):
    mesh = plsc.VectorSubcoreMesh(core_axis_name="c", subcore_axis_name="s")
    out_t = jax.ShapeDtypeStruct((B, N), jnp.float32)
    fn = functools.partial(
        pl.kernel,
        mesh=mesh,
        out_type=(out_t,) * 6,
        scratch_types=[
            pltpu.VMEM((N,), jnp.float32),
            pltpu.VMEM((2, CHUNK), jnp.int32),
            pltpu.VMEM((2, CHUNK), jnp.float32),
            pltpu.SemaphoreType.DMA,
            pltpu.SemaphoreType.DMA,
            pltpu.SemaphoreType.DMA,
            pltpu.SemaphoreType.DMA,
            pltpu.SemaphoreType.DMA,
        ],
        compiler_params=pltpu.CompilerParams(needs_layout_passes=False),
    )(_sc_body)
    return fn(qt, idx)


def kernel(fea_f, fea_fenzi, fea_fenmu, y, idx, memory_fringe, memory_fenzi,
           memory_fenmu):
    del y
    idx = idx.astype(jnp.int32)
    f_z = jnp.concatenate([fea_f, fea_fenmu], axis=0)
    f_m = jnp.concatenate([fea_f, fea_fenzi], axis=0)
    f_r = jnp.concatenate([fea_fenzi, fea_fenmu], axis=0)

    qt = _compute_qt(f_z, f_m, f_r, memory_fenzi.T, memory_fenmu.T,
                     memory_fringe.T)
    f_fenzi, f_fenmu, fenzi_f, fenzi_fenmu, fenmu_f, fenmu_fenzi = (
        _sc_gather(qt, idx))

    return (f_fenzi[..., None], f_fenmu[..., None], fenzi_f[..., None],
            fenzi_fenmu[..., None], fenmu_f[..., None],
            fenmu_fenzi[..., None])


# speedup vs baseline: 126.6715x; 1.5171x over previous
import functools

import jax
import jax.numpy as jnp
from jax import lax
from jax.experimental import pallas as pl
from jax.experimental.pallas import tpu as pltpu
from jax.experimental.pallas import tpu_sc as plsc

B = 16
N = 65536
FEAT = 64
INV_T = 1.0 / 0.07

NUM_WORKERS = 32
ROWS = 6 * B
ROUNDS = ROWS // NUM_WORKERS
CHUNK = 8192
N_CHUNKS = N // CHUNK


def _qt_kernel(fz_ref, fm_ref, fr_ref, bz_ref, bm_ref, br_ref, out_ref):
    fz = fz_ref[...] * INV_T
    fm = fm_ref[...] * INV_T
    fr = fr_ref[...] * INV_T
    dims = (((1,), (0,)), ((), ()))
    blk = bz_ref.shape[1]
    qz = lax.dot_general(
        fz, bz_ref[...], dims, preferred_element_type=jnp.float32)
    qm = lax.dot_general(
        fm, bm_ref[...], dims, preferred_element_type=jnp.float32)
    qr = lax.dot_general(
        fr, br_ref[...], dims, preferred_element_type=jnp.float32)
    out_ref[0:32] = qz.reshape(32, blk // 128, 128)
    out_ref[32:64] = qm.reshape(32, blk // 128, 128)
    out_ref[64:96] = qr.reshape(32, blk // 128, 128)


def _compute_qt(f_z, f_m, f_r, bank_zt, bank_mt, bank_rt):
    blk = 4096
    grid = (N // blk,)
    return pl.pallas_call(
        _qt_kernel,
        grid=grid,
        in_specs=[
            pl.BlockSpec((32, FEAT), lambda i: (0, 0)),
            pl.BlockSpec((32, FEAT), lambda i: (0, 0)),
            pl.BlockSpec((32, FEAT), lambda i: (0, 0)),
            pl.BlockSpec((FEAT, blk), lambda i: (0, i)),
            pl.BlockSpec((FEAT, blk), lambda i: (0, i)),
            pl.BlockSpec((FEAT, blk), lambda i: (0, i)),
        ],
        out_specs=pl.BlockSpec((ROWS, blk // 128, 128), lambda i: (0, i, 0)),
        out_shape=jax.ShapeDtypeStruct((ROWS, N // 128, 128), jnp.float32),
    )(f_z, f_m, f_r, bank_zt, bank_mt, bank_rt)


def _sc_body(qt_hbm, idx_hbm, o0, o1, o2, o3, o4, o5,
             q_v, idx_v, o_v, sem_q, sem_i0, sem_i1, sem_o0, sem_o1):
    c = lax.axis_index("c")
    s = lax.axis_index("s")
    wid = s * 2 + c
    b = lax.rem(wid, B)
    hi = wid >= B
    outs_lo = (o0, o1, o2)
    outs_hi = (o5, o3, o4)
    idx_sems = (sem_i0, sem_i1)
    out_sems = (sem_o0, sem_o1)

    def idx_copy(ch, bf):
        return pltpu.make_async_copy(
            idx_hbm.at[b, pl.ds(ch * CHUNK, CHUNK)], idx_v.at[bf],
            idx_sems[bf])

    def out_drain(t, ch, bf):
        col = pl.ds(ch * (CHUNK // 128), CHUNK // 128)

        @pl.when(jnp.logical_not(hi))
        def _():
            pltpu.make_async_copy(o_v.at[bf], outs_lo[t].at[b, col],
                                  out_sems[bf]).wait()

        @pl.when(hi)
        def _():
            pltpu.make_async_copy(o_v.at[bf], outs_hi[t].at[b, col],
                                  out_sems[bf]).wait()

    out_pending = [None, None]
    for t in range(ROUNDS):
        r = t * NUM_WORKERS + wid
        pltpu.sync_copy(qt_hbm.at[r], q_v)
        for ch in range(N_CHUNKS):
            bf = ch % 2
            if ch == 0:
                idx_copy(0, 0).start()
            cur = idx_copy(ch, bf)
            cur.wait()
            if ch + 1 < N_CHUNKS:
                idx_copy(ch + 1, (ch + 1) % 2).start()
            if out_pending[bf] is not None:
                out_drain(*out_pending[bf], bf)

            @plsc.parallel_loop(0, CHUNK, 16, unroll=8)
            def _gather(i):
                iv = idx_v[bf, pl.ds(i, 16)]
                o_v[bf, lax.shift_right_logical(i, 7),
                    pl.ds(lax.bitwise_and(i, 127), 16)] = plsc.load_gather(
                    q_v, [lax.shift_right_logical(iv, 7),
                          lax.bitwise_and(iv, 127)])

            col = pl.ds(ch * (CHUNK // 128), CHUNK // 128)

            @pl.when(jnp.logical_not(hi))
            def _():
                pltpu.async_copy(o_v.at[bf], outs_lo[t].at[b, col],
                                 out_sems[bf])

            @pl.when(hi)
            def _():
                pltpu.async_copy(o_v.at[bf], outs_hi[t].at[b, col],
                                 out_sems[bf])

            out_pending[bf] = (t, ch)
    for bf in range(2):
        if out_pending[bf] is not None:
            out_drain(*out_pending[bf], bf)


def _sc_gather(qt, idx):
    mesh = plsc.VectorSubcoreMesh(core_axis_name="c", subcore_axis_name="s")
    out_t = jax.ShapeDtypeStruct((B, N // 128, 128), jnp.float32)
    fn = functools.partial(
        pl.kernel,
        mesh=mesh,
        out_type=(out_t,) * 6,
        scratch_types=[
            pltpu.VMEM((N // 128, 128), jnp.float32),
            pltpu.VMEM((2, CHUNK), jnp.int32),
            pltpu.VMEM((2, CHUNK // 128, 128), jnp.float32),
            pltpu.SemaphoreType.DMA,
            pltpu.SemaphoreType.DMA,
            pltpu.SemaphoreType.DMA,
            pltpu.SemaphoreType.DMA,
            pltpu.SemaphoreType.DMA,
        ],
        compiler_params=pltpu.CompilerParams(needs_layout_passes=False),
    )(_sc_body)
    return fn(qt, idx)


def kernel(fea_f, fea_fenzi, fea_fenmu, y, idx, memory_fringe, memory_fenzi,
           memory_fenmu):
    del y
    idx = idx.astype(jnp.int32)
    f_z = jnp.concatenate([fea_f, fea_fenmu], axis=0)
    f_m = jnp.concatenate([fea_f, fea_fenzi], axis=0)
    f_r = jnp.concatenate([fea_fenzi, fea_fenmu], axis=0)

    qt = _compute_qt(f_z, f_m, f_r, memory_fenzi.T, memory_fenmu.T,
                     memory_fringe.T)
    outs = _sc_gather(qt, idx)
    return tuple(o.reshape(B, N, 1) for o in outs)
